# baseline (speedup 1.0000x reference)
import functools

import jax
import jax.numpy as jnp
from jax import lax
from jax.experimental import pallas as pl
from jax.experimental.pallas import tpu as pltpu
from jax.experimental.pallas import tpu_sc as plsc

N = 10000
E = 320000
D = 128
H = 128

NC = 2
NS = 16

CW = 16
C2 = 88

ZR = 624
ZTAIL_BASE = ZR * 15
ZTAIL = N - ZTAIL_BASE - ZR


def _zero_and_barrier(zeros_hbm, acc, s):
    pltpu.sync_copy(zeros_hbm.at[pl.ds(s * ZR, ZR)], acc.at[pl.ds(s * ZR, ZR)])

    @pl.when(s == NS - 1)
    def _():
        pltpu.sync_copy(zeros_hbm.at[pl.ds(ZTAIL_BASE + ZR, ZTAIL)],
                        acc.at[pl.ds(ZTAIL_BASE + ZR, ZTAIL)])

    plsc.subcore_barrier()


def _writeback(acc, out_hbm, c, s):
    pltpu.sync_copy(acc.at[pl.ds(s * ZR, ZR)], out_hbm.at[c, pl.ds(s * ZR, ZR)])

    @pl.when(s == NS - 1)
    def _():
        pltpu.sync_copy(acc.at[pl.ds(ZTAIL_BASE + ZR, ZTAIL)],
                        out_hbm.at[c, pl.ds(ZTAIL_BASE + ZR, ZTAIL)])


def _emit_sweep(n, nb, ce, src_ix, dst_ix, tbl, acc, srcb, dstb, rows, sems,
                cnt=None):
    nbb = 2 * nb
    dd = 2 * nbb
    gsems = sems[:nb]
    ssems = sems[nb:2 * nb]
    isems = sems[2 * nb:2 * nb + nbb]
    dsems = sems[2 * nb + nbb:6 * nb]
    if cnt is not None:
        acc_cnt, ones_ref = cnt
        csems = sems[6 * nb:]

    def issue_idx(j, bb):
        sb, db = bb % nbb, bb % dd
        pltpu.async_copy(src_ix(j), srcb.at[sb], isems[sb])
        pltpu.async_copy(dst_ix(j), dstb.at[db], dsems[sb])

    def issue_gather(j, bb):
        b, sb, rb = bb % nb, bb % nbb, bb % nbb
        pltpu.make_async_copy(src_ix(j), srcb.at[sb], isems[sb]).wait()
        pltpu.async_copy(tbl.at[srcb.at[sb]], rows.at[rb], gsems[b])

    def wait_scatter(bb):
        b, rb = bb % nb, bb % nbb
        pltpu.make_async_copy(rows.at[rb], acc.at[dstb.at[0]], ssems[b]).wait()
        if cnt is not None:
            pltpu.make_async_copy(ones_ref, acc_cnt.at[dstb.at[0]], csems[b]).wait()

    def process(j, bb, wait_prev, pf_idx, pf_gather):
        b, sb, rb, db = bb % nb, bb % nbb, bb % nbb, bb % dd
        if wait_prev:
            wait_scatter(bb + nb)
        pltpu.make_async_copy(tbl.at[srcb.at[sb]], rows.at[rb], gsems[b]).wait()
        pltpu.make_async_copy(dst_ix(j), dstb.at[db], dsems[sb]).wait()
        pltpu.async_copy(rows.at[rb], acc.at[dstb.at[db]], ssems[b], add=True)
        if cnt is not None:
            pltpu.async_copy(ones_ref, acc_cnt.at[dstb.at[db]], csems[b], add=True)
        if pf_idx:
            issue_idx(j + nbb, bb + nbb)
        if pf_gather:
            issue_gather(j + nb, bb + nb)

    ngrp = n // dd
    for j in range(nbb):
        issue_idx(j, j)
    for j in range(nb):
        issue_gather(j, j)
    for bb in range(dd):
        process(bb, bb, bb >= nb, bb + nbb < n, bb + nb < n)

    def grp(g, _):
        for bb in range(dd):
            process(g * dd + bb, bb, True, True, True)
        return _

    lax.fori_loop(1, ngrp - 1, grp, 0)
    for bb in range(dd):
        j = (ngrp - 1) * dd + bb
        process(j, bb, True, j + nbb < n, j + nb < n)
    for t in range(n - ngrp * dd):
        j = ngrp * dd + t
        process(j, t, True, j + nbb < n, j + nb < n)
    for j in range(n - nb, n):
        wait_scatter(j % nbb)


def _sweep_scratch(nb, ce, width):
    return [
        pltpu.VMEM((2 * nb, ce), jnp.int32),
        pltpu.VMEM((4 * nb, ce), jnp.int32),
        pltpu.VMEM((2 * nb, ce, width), jnp.float32),
    ] + [pltpu.SemaphoreType.DMA] * (6 * nb)


NB1, CE1 = 3, 40
NB2, CE2 = 4, 80


def _sc_pass1(x, edges3, zeros1, zerosc, ones16):
    n_chunks = E // (NC * NS) // CE1

    @functools.partial(
        pl.kernel,
        out_type=[jax.ShapeDtypeStruct((NC, N, D), jnp.float32),
                  jax.ShapeDtypeStruct((NC, N, CW), jnp.float32)],
        mesh=plsc.VectorSubcoreMesh(core_axis_name="c", subcore_axis_name="s"),
        compiler_params=pltpu.CompilerParams(use_tc_tiling_on_sc=False),
        scratch_types=_sweep_scratch(NB1, CE1, D)
        + [pltpu.SemaphoreType.DMA] * NB1
        + [pltpu.VMEM((CE1, CW), jnp.float32),
           pltpu.VMEM_SHARED((N, D), jnp.float32),
           pltpu.VMEM_SHARED((N, CW), jnp.float32)],
    )
    def k(x_h, e3_h, zeros_h, zc_h, ones_h, out_h, outc_h,
          srcb, dstb, rows, *rest):
        sems = list(rest[:7 * NB1])
        onesb, acc, acc_cnt = rest[7 * NB1], rest[7 * NB1 + 1], rest[7 * NB1 + 2]
        c = lax.axis_index("c")
        s = lax.axis_index("s")
        pltpu.sync_copy(ones_h, onesb)
        pltpu.sync_copy(zc_h.at[pl.ds(s * ZR, ZR)], acc_cnt.at[pl.ds(s * ZR, ZR)])

        @pl.when(s == NS - 1)
        def _():
            pltpu.sync_copy(zc_h.at[pl.ds(ZTAIL_BASE + ZR, ZTAIL)],
                            acc_cnt.at[pl.ds(ZTAIL_BASE + ZR, ZTAIL)])

        _zero_and_barrier(zeros_h, acc, s)
        tid = c * NS + s
        edge_base = tid * (E // (NC * NS))
        _emit_sweep(
            n_chunks, NB1, CE1,
            lambda j: e3_h.at[0, pl.ds(edge_base + j * CE1, CE1)],
            lambda j: e3_h.at[2, pl.ds(edge_base + j * CE1, CE1)],
            x_h, acc, srcb, dstb, rows, sems, cnt=(acc_cnt, onesb))
        plsc.subcore_barrier()
        _writeback(acc, out_h, c, s)
        pltpu.sync_copy(acc_cnt.at[pl.ds(s * ZR, ZR)], outc_h.at[c, pl.ds(s * ZR, ZR)])

        @pl.when(s == NS - 1)
        def _():
            pltpu.sync_copy(acc_cnt.at[pl.ds(ZTAIL_BASE + ZR, ZTAIL)],
                            outc_h.at[c, pl.ds(ZTAIL_BASE + ZR, ZTAIL)])

    return k(x, edges3, zeros1, zerosc, ones16)


def _sc_pass2(z_flat, edges3, zeros2):
    n_chunks = E // NS // CE2

    @functools.partial(
        pl.kernel,
        out_type=jax.ShapeDtypeStruct((NC, N, C2), jnp.float32),
        mesh=plsc.VectorSubcoreMesh(core_axis_name="c", subcore_axis_name="s"),
        compiler_params=pltpu.CompilerParams(use_tc_tiling_on_sc=False),
        scratch_types=_sweep_scratch(NB2, CE2, C2) + [pltpu.VMEM_SHARED((N, C2), jnp.float32)],
    )
    def k(z_h, e3_h, zeros_h, out_h, srcb, dstb, rows, *rest):
        sems, acc = list(rest[:6 * NB2]), rest[6 * NB2]
        c = lax.axis_index("c")
        s = lax.axis_index("s")
        edge_base = s * (E // NS)
        _zero_and_barrier(zeros_h, acc, s)
        _emit_sweep(
            n_chunks, NB2, CE2,
            lambda j: e3_h.at[c, pl.ds(edge_base + j * CE2, CE2)],
            lambda j: e3_h.at[2, pl.ds(edge_base + j * CE2, CE2)],
            z_h, acc, srcb, dstb, rows, sems)
        plsc.subcore_barrier()
        _writeback(acc, out_h, c, s)

    return k(z_flat, edges3, zeros2)


RB = 2000

OA, OS, OG = 129, 27, 11
OZ = OA + OS + OG


def _stage_b_body(s1p, cntp, xb, wla, bla, wra, wls, bls, wrs, wlg, blg, wrg,
                  woa, wos, wog, ba, ra, bs, rs, bg, rg, ha, hs, hg, z2, w_out, rcnt):
    cnt = cntp[0][:, :1] + cntp[1][:, :1]
    rc = 1.0 / jnp.maximum(cnt, 1.0)
    agg = (s1p[0] + s1p[1]) * rc
    x = xb[...]

    def head(wl, bl, wr):
        return (jnp.dot(agg, wl[...], preferred_element_type=jnp.float32)
                + bl[...]
                + jnp.dot(x, wr[...], preferred_element_type=jnp.float32))

    h_a = head(wla, bla, wra)
    h_s = head(wls, bls, wrs)
    h_g = head(wlg, blg, wrg)
    ha[...] = h_a
    hs[...] = h_s
    hg[...] = h_g
    e_a = jnp.maximum(h_a, 0.0)
    e_s = jnp.maximum(h_s, 0.0)
    e_g = jnp.maximum(h_g, 0.0)
    pad = jnp.zeros((RB, 2 * C2 - OZ), jnp.float32)
    z = jnp.concatenate(
        [jnp.dot(e_a, woa[...], preferred_element_type=jnp.float32),
         jnp.dot(e_s, wos[...], preferred_element_type=jnp.float32),
         jnp.dot(e_g, wog[...], preferred_element_type=jnp.float32),
         pad],
        axis=1)
    z2[0] = z[:, :C2]
    z2[1] = z[:, C2:]
    w_out[...] = jnp.concatenate(
        [jnp.dot(e_a, ra[...], preferred_element_type=jnp.float32) + ba[...],
         jnp.dot(e_s, rs[...], preferred_element_type=jnp.float32) + bs[...],
         jnp.dot(e_g, rg[...], preferred_element_type=jnp.float32) + bg[...],
         pad],
        axis=1)
    rcnt[...] = jnp.broadcast_to(rc, (RB, 8))


def _tc_stage_b(s1p, cntp, x, wla, bla, wra, wls, bls, wrs, wlg, blg, wrg,
                woa, wos, wog, ba, ra, bs, rs, bg, rg):
    grid = (N // RB,)
    full = lambda shape: pl.BlockSpec(shape, lambda i: (0,) * len(shape))
    row = lambda w: pl.BlockSpec((RB, w), lambda i: (i, 0))
    return pl.pallas_call(
        _stage_b_body,
        grid=grid,
        in_specs=[
            pl.BlockSpec((NC, RB, D), lambda i: (0, i, 0)),
            pl.BlockSpec((NC, RB, CW), lambda i: (0, i, 0)),
            row(D),
            full((D, H)), full((1, H)), full((D, H)),
            full((D, H)), full((1, H)), full((D, H)),
            full((D, H)), full((1, H)), full((D, H)),
            full((H, OA)), full((H, OS)), full((H, OG)),
            full((1, OA)), full((H, OA)),
            full((1, OS)), full((H, OS)),
            full((1, OG)), full((H, OG)),
        ],
        out_specs=[
            row(H), row(H), row(H),
            pl.BlockSpec((NC, RB, C2), lambda i: (0, i, 0)),
            row(2 * C2),
            row(8),
        ],
        out_shape=[
            jax.ShapeDtypeStruct((N, H), jnp.float32),
            jax.ShapeDtypeStruct((N, H), jnp.float32),
            jax.ShapeDtypeStruct((N, H), jnp.float32),
            jax.ShapeDtypeStruct((NC, N, C2), jnp.float32),
            jax.ShapeDtypeStruct((N, 2 * C2), jnp.float32),
            jax.ShapeDtypeStruct((N, 8), jnp.float32),
        ],
    )(s1p, cntp, x, wla, bla, wra, wls, bls, wrs, wlg, blg, wrg, woa, wos, wog,
      ba, ra, bs, rs, bg, rg)


def _log_softmax(xo):
    m = jnp.max(xo, axis=1, keepdims=True)
    e = jnp.exp(xo - m)
    return xo - m - jnp.log(jnp.sum(e, axis=1, keepdims=True))


def _stage_d_body(s2, rcnt, wb, ya, ys, yg):
    rc = rcnt[:, :1]
    xo = jnp.concatenate([s2[0], s2[1]], axis=1) * rc + wb[...]
    ya[...] = _log_softmax(xo[:, :OA])
    ys[...] = _log_softmax(xo[:, OA:OA + OS])
    yg[...] = _log_softmax(xo[:, OA + OS:OZ])


def _tc_stage_d(s2, rcnt, w):
    grid = (N // RB,)
    row = lambda wd: pl.BlockSpec((RB, wd), lambda i: (i, 0))
    return pl.pallas_call(
        _stage_d_body,
        grid=grid,
        in_specs=[
            pl.BlockSpec((NC, RB, C2), lambda i: (0, i, 0)),
            row(8), row(2 * C2),
        ],
        out_specs=[row(OA), row(OS), row(OG)],
        out_shape=[
            jax.ShapeDtypeStruct((N, OA), jnp.float32),
            jax.ShapeDtypeStruct((N, OS), jnp.float32),
            jax.ShapeDtypeStruct((N, OG), jnp.float32),
        ],
    )(s2, rcnt, w)


def kernel(x, edge_index,
           Wl1_artist, bl1_artist, Wr1_artist, Wlo_artist, blo_artist, Wro_artist,
           Wl1_style, bl1_style, Wr1_style, Wlo_style, blo_style, Wro_style,
           Wl1_genre, bl1_genre, Wr1_genre, Wlo_genre, blo_genre, Wro_genre):
    src = edge_index[0].astype(jnp.int32)
    dst = edge_index[1].astype(jnp.int32)
    edges3 = jnp.stack([src, src + N, dst])

    zeros1 = jnp.zeros((N, D), jnp.float32)
    zerosc = jnp.zeros((N, CW), jnp.float32)
    zeros2 = jnp.zeros((N, C2), jnp.float32)
    ones16 = jnp.ones((CE1, CW), jnp.float32)

    s1p, cntp = _sc_pass1(x, edges3, zeros1, zerosc, ones16)

    ha, hs, hg, z2, w, rcnt = _tc_stage_b(
        s1p, cntp, x,
        Wl1_artist, bl1_artist.reshape(1, -1), Wr1_artist,
        Wl1_style, bl1_style.reshape(1, -1), Wr1_style,
        Wl1_genre, bl1_genre.reshape(1, -1), Wr1_genre,
        Wlo_artist, Wlo_style, Wlo_genre,
        blo_artist.reshape(1, -1), Wro_artist,
        blo_style.reshape(1, -1), Wro_style,
        blo_genre.reshape(1, -1), Wro_genre)

    s2 = _sc_pass2(z2.reshape(2 * N, C2), edges3, zeros2)

    ya, ys, yg = _tc_stage_d(s2, rcnt, w)

    return (ha, ya, hs, ys, hg, yg)
